# baseline (speedup 1.0000x reference)
import functools

import jax
import jax.numpy as jnp
from jax import lax
from jax.experimental import pallas as pl
from jax.experimental.pallas import tpu as pltpu
from jax.experimental.pallas import tpu_sc as plsc

_B = 128
_N = 4096
_T = 2.0
_NC = 2
_NS = 16
_NW = _NC * _NS
_BPW = _B // _NW
_NA = 9
_VPR = _N // 16
_NSC = 3
_NBLK = 16
_BLK = _N // _NBLK
_NEG = float("-inf")


@functools.partial(
    pl.kernel,
    mesh=plsc.VectorSubcoreMesh(core_axis_name="c", subcore_axis_name="s"),
    out_type=[
        jax.ShapeDtypeStruct((_B, 16), jnp.float32),
        jax.ShapeDtypeStruct((_NW, 16), jnp.float32),
    ],
    scratch_types=[
        pltpu.VMEM((_BPW, _N), jnp.float32),
        pltpu.VMEM((_BPW, _N), jnp.float32),
        pltpu.VMEM((_B,), jnp.int32),
        pltpu.VMEM((_BPW, 16), jnp.float32),
        pltpu.VMEM((1, 16), jnp.float32),
        pltpu.SemaphoreType.DMA,
        pltpu.SemaphoreType.DMA,
    ],
    compiler_params=pltpu.CompilerParams(needs_layout_passes=False,
                                         skip_device_barrier=True),
)
def _sc_stage(o1, o2, o3, tgt_hbm,
              marg_out, part_out,
              buf0, buf1, tgt_v, marg_v, pm_v, sem0, sem1):
    refs = [o1, o2, o3]
    wid = lax.axis_index("c") * _NS + lax.axis_index("s")
    b0 = wid * _BPW

    pltpu.sync_copy(tgt_hbm, tgt_v)

    bufs = [buf0, buf1]
    sems = [sem0, sem1]

    def start(a):
        return pltpu.async_copy(refs[a].at[pl.ds(b0, _BPW)], bufs[a % 2],
                                sems[a % 2])

    pending = start(0)
    pm = jnp.float32(_NEG)
    neg_vec = jnp.full((16,), _NEG)
    lane = lax.iota(jnp.int32, 16)
    zero_vec = jnp.zeros((16,), jnp.float32)

    for bi in range(_BPW):
        marg_v[bi, :] = zero_vec

    for a in range(_NSC):
        buf = bufs[a % 2]
        cur = pending
        if a + 1 < _NSC:
            pending = start(a + 1)
        cur.wait()

        def row_body(bi, pm, buf=buf, a=a):
            bi_vec = jnp.full((16,), bi, jnp.int32)
            t_all = plsc.load_gather(tgt_v, [jnp.full((16,), b0 + bi, jnp.int32)])

            def mbody(i, accs):
                a0, a1, a2, a3 = accs
                base = i * 16
                acc = [a0, a1, a2, a3]
                for u in range(16):
                    v = buf[bi, pl.ds(base + u * 16, 16)]
                    acc[u % 4] = jnp.maximum(acc[u % 4], v)
                return tuple(acc)

            a0, a1, a2, a3 = plsc.parallel_loop(
                0, _VPR, step=16, unroll=2,
                carry=(neg_vec, neg_vec, neg_vec, neg_vec))(mbody)
            macc = jnp.maximum(jnp.maximum(a0, a1), jnp.maximum(a2, a3))
            m = jnp.max(macc)
            v_t = plsc.load_gather(buf, [bi_vec, t_all])[0]

            def fix():
                l0_vec = t_all & 15
                col_base = l0_vec + lane * 16

                def fbody(i, caccs):
                    out = list(caccs)
                    for u in range(4):
                        idx = col_base + (i * 4 + u) * 256
                        g = plsc.load_gather(buf, [bi_vec, idx])
                        out[u] = jnp.maximum(
                            out[u], jnp.where(idx == t_all, _NEG, g))
                    return tuple(out)

                c0, c1, c2, c3 = lax.fori_loop(
                    0, 4, fbody, (neg_vec, neg_vec, neg_vec, neg_vec))
                call = jnp.maximum(jnp.maximum(c0, c1), jnp.maximum(c2, c3))
                macc_o = jnp.where(lane == l0_vec, _NEG, macc)
                me = jnp.maximum(jnp.max(macc_o), jnp.max(call))
                return m - me

            margin = lax.cond(v_t >= m, fix, lambda: jnp.float32(0.0))

            mv = marg_v[bi, :]
            marg_v[bi, :] = jnp.where(lane == a, margin, mv)
            return jnp.maximum(pm, m)

        pm = lax.fori_loop(0, _BPW, row_body, pm)

    pm_v[0, :] = jnp.full((16,), pm)
    pltpu.sync_copy(marg_v, marg_out.at[pl.ds(b0, _BPW)])
    pltpu.sync_copy(pm_v, part_out.at[pl.ds(wid, 1)])


def _tc_marg_body(o4r, o5r, o6r, o7r, o8r, mimr, tgtr, marg_ref, pm_ref,
                  acc_ref):
    i = pl.program_id(0)
    tcol = tgtr[...]
    col = lax.broadcasted_iota(jnp.int32, (_B, _BLK), 1) + i * _BLK
    hit = col == tcol
    neg = jnp.float32(_NEG)
    ms, mes = [], []
    for r in (o4r, o5r, o6r, o7r, o8r, mimr):
        x = r[...]
        ms.append(jnp.max(x, axis=1, keepdims=True))
        mes.append(jnp.max(jnp.where(hit, neg, x), axis=1, keepdims=True))
    new = jnp.concatenate(
        ms + mes + [jnp.full((_B, 16 - 2 * 6), _NEG)], axis=1)
    prev = jnp.where(i == 0, jnp.full((_B, 16), _NEG), acc_ref[...])
    upd = jnp.maximum(prev, new)
    acc_ref[...] = upd

    @pl.when(i == _NBLK - 1)
    def _():
        m6 = upd[:, 0:6]
        me6 = upd[:, 6:12]
        marg_ref[...] = jnp.concatenate(
            [jnp.zeros((_B, _NSC)), m6 - me6, jnp.zeros((_B, 16 - _NSC - 6))],
            axis=1)
        pm_ref[...] = jnp.full((1, 1), jnp.max(upd[:, 0:5]))


_tc_marg = pl.pallas_call(
    _tc_marg_body,
    grid=(_NBLK,),
    in_specs=[pl.BlockSpec((_B, _BLK), lambda i: (0, i))] * 6
    + [pl.BlockSpec((_B, 1), lambda i: (0, 0))],
    out_specs=(pl.BlockSpec((_B, 16), lambda i: (0, 0)),
               pl.BlockSpec((1, 1), lambda i: (0, 0))),
    out_shape=(jax.ShapeDtypeStruct((_B, 16), jnp.float32),
               jax.ShapeDtypeStruct((1, 1), jnp.float32)),
    scratch_shapes=[pltpu.VMEM((_B, 16), jnp.float32)],
)


def _fin_body(msc, mtc, psc, ptc, thr_ref, mx_ref):
    margins = msc[...] + mtc[...]
    lanes = lax.broadcasted_iota(jnp.int32, (_B, 16), 1)
    valid = lanes < _NA
    logits = margins * jnp.float32(1.0 / _T)
    mrow = jnp.max(jnp.where(valid, logits, jnp.float32(-1e30)),
                   axis=1, keepdims=True)
    e = jnp.where(valid, jnp.exp(logits - mrow), jnp.float32(0.0))
    thr_ref[...] = (e / jnp.sum(e, axis=1, keepdims=True))[:, :_NA]
    mx_ref[...] = jnp.full((1, 1),
                           jnp.maximum(jnp.max(psc[...]), jnp.max(ptc[...])))


_fin = pl.pallas_call(
    _fin_body,
    out_shape=(jax.ShapeDtypeStruct((_B, _NA), jnp.float32),
               jax.ShapeDtypeStruct((1, 1), jnp.float32)),
)


def kernel(outputs1, outputs2, outputs3, outputs4, outputs5, outputs6,
           outputs7, outputs8, mimic, targets, n_test):
    tgt32 = targets.astype(jnp.int32)
    marg_sc, part_sc = _sc_stage(outputs1, outputs2, outputs3, tgt32)
    marg_tc, pm_tc = _tc_marg(outputs4, outputs5, outputs6, outputs7,
                              outputs8, mimic, tgt32.reshape(_B, 1))
    thr, mx = _fin(marg_sc, marg_tc, part_sc, pm_tc)
    return mx.reshape(()), thr


# speedup vs baseline: 1.0869x; 1.0869x over previous
import functools

import jax
import jax.numpy as jnp
from jax import lax
from jax.experimental import pallas as pl
from jax.experimental.pallas import tpu as pltpu
from jax.experimental.pallas import tpu_sc as plsc

_B = 128
_N = 4096
_T = 2.0
_NC = 2
_NS = 16
_NW = _NC * _NS
_BPW = _B // _NW
_NA = 9
_VPR = _N // 16
_NSC = 4
_NBLK = 8
_BLK = _N // _NBLK
_NEG = float("-inf")


@functools.partial(
    pl.kernel,
    mesh=plsc.VectorSubcoreMesh(core_axis_name="c", subcore_axis_name="s"),
    out_type=[
        jax.ShapeDtypeStruct((_B, 16), jnp.float32),
        jax.ShapeDtypeStruct((_NW, 16), jnp.float32),
    ],
    scratch_types=[
        pltpu.VMEM((_BPW, _N), jnp.float32),
        pltpu.VMEM((_BPW, _N), jnp.float32),
        pltpu.VMEM((_B,), jnp.int32),
        pltpu.VMEM((_BPW, 16), jnp.float32),
        pltpu.VMEM((1, 16), jnp.float32),
        pltpu.SemaphoreType.DMA,
        pltpu.SemaphoreType.DMA,
    ],
    compiler_params=pltpu.CompilerParams(needs_layout_passes=False,
                                         skip_device_barrier=True),
)
def _sc_stage(o1, o2, o3, o4, tgt_hbm,
              marg_out, part_out,
              buf0, buf1, tgt_v, marg_v, pm_v, sem0, sem1):
    refs = [o1, o2, o3, o4]
    wid = lax.axis_index("c") * _NS + lax.axis_index("s")
    b0 = wid * _BPW

    pltpu.sync_copy(tgt_hbm, tgt_v)

    bufs = [buf0, buf1]
    sems = [sem0, sem1]

    def start(a):
        return pltpu.async_copy(refs[a].at[pl.ds(b0, _BPW)], bufs[a % 2],
                                sems[a % 2])

    pending = start(0)
    pm = jnp.float32(_NEG)
    neg_vec = jnp.full((16,), _NEG)
    lane = lax.iota(jnp.int32, 16)
    zero_vec = jnp.zeros((16,), jnp.float32)

    for bi in range(_BPW):
        marg_v[bi, :] = zero_vec

    for a in range(_NSC):
        buf = bufs[a % 2]
        cur = pending
        if a + 1 < _NSC:
            pending = start(a + 1)
        cur.wait()

        def row_body(bi, pm, buf=buf, a=a):
            bi_vec = jnp.full((16,), bi, jnp.int32)
            t_all = plsc.load_gather(tgt_v, [jnp.full((16,), b0 + bi, jnp.int32)])

            def mbody(i, accs):
                a0, a1, a2, a3 = accs
                base = i * 16
                acc = [a0, a1, a2, a3]
                for u in range(16):
                    v = buf[bi, pl.ds(base + u * 16, 16)]
                    acc[u % 4] = jnp.maximum(acc[u % 4], v)
                return tuple(acc)

            a0, a1, a2, a3 = plsc.parallel_loop(
                0, _VPR, step=16, unroll=2,
                carry=(neg_vec, neg_vec, neg_vec, neg_vec))(mbody)
            macc = jnp.maximum(jnp.maximum(a0, a1), jnp.maximum(a2, a3))
            m = jnp.max(macc)
            v_t = plsc.load_gather(buf, [bi_vec, t_all])[0]

            def fix():
                l0_vec = t_all & 15
                col_base = l0_vec + lane * 16

                def fbody(i, caccs):
                    out = list(caccs)
                    for u in range(4):
                        idx = col_base + (i * 4 + u) * 256
                        g = plsc.load_gather(buf, [bi_vec, idx])
                        out[u] = jnp.maximum(
                            out[u], jnp.where(idx == t_all, _NEG, g))
                    return tuple(out)

                c0, c1, c2, c3 = lax.fori_loop(
                    0, 4, fbody, (neg_vec, neg_vec, neg_vec, neg_vec))
                call = jnp.maximum(jnp.maximum(c0, c1), jnp.maximum(c2, c3))
                macc_o = jnp.where(lane == l0_vec, _NEG, macc)
                me = jnp.maximum(jnp.max(macc_o), jnp.max(call))
                return m - me

            margin = lax.cond(v_t >= m, fix, lambda: jnp.float32(0.0))

            mv = marg_v[bi, :]
            marg_v[bi, :] = jnp.where(lane == a, margin, mv)
            return jnp.maximum(pm, m)

        pm = lax.fori_loop(0, _BPW, row_body, pm)

    pm_v[0, :] = jnp.full((16,), pm)
    pltpu.sync_copy(marg_v, marg_out.at[pl.ds(b0, _BPW)])
    pltpu.sync_copy(pm_v, part_out.at[pl.ds(wid, 1)])


def _tc_marg_body(o5r, o6r, o7r, o8r, mimr, tgtr, marg_ref, pm_ref,
                  acc_ref):
    i = pl.program_id(0)
    tcol = tgtr[...]
    col = lax.broadcasted_iota(jnp.int32, (_B, _BLK), 1) + i * _BLK
    hit = col == tcol
    neg = jnp.float32(_NEG)
    ms, mes = [], []
    for r in (o5r, o6r, o7r, o8r, mimr):
        x = r[...]
        ms.append(jnp.max(x, axis=1, keepdims=True))
        mes.append(jnp.max(jnp.where(hit, neg, x), axis=1, keepdims=True))
    new = jnp.concatenate(
        ms + mes + [jnp.full((_B, 16 - 2 * 5), _NEG)], axis=1)
    prev = jnp.where(i == 0, jnp.full((_B, 16), _NEG), acc_ref[...])
    upd = jnp.maximum(prev, new)
    acc_ref[...] = upd

    @pl.when(i == _NBLK - 1)
    def _():
        m5 = upd[:, 0:5]
        me5 = upd[:, 5:10]
        marg_ref[...] = jnp.concatenate(
            [jnp.zeros((_B, _NSC)), m5 - me5, jnp.zeros((_B, 16 - _NSC - 5))],
            axis=1)
        pm_ref[...] = jnp.full((1, 1), jnp.max(upd[:, 0:4]))


_tc_marg = pl.pallas_call(
    _tc_marg_body,
    grid=(_NBLK,),
    in_specs=[pl.BlockSpec((_B, _BLK), lambda i: (0, i))] * 5
    + [pl.BlockSpec((_B, 1), lambda i: (0, 0))],
    out_specs=(pl.BlockSpec((_B, 16), lambda i: (0, 0)),
               pl.BlockSpec((1, 1), lambda i: (0, 0))),
    out_shape=(jax.ShapeDtypeStruct((_B, 16), jnp.float32),
               jax.ShapeDtypeStruct((1, 1), jnp.float32)),
    scratch_shapes=[pltpu.VMEM((_B, 16), jnp.float32)],
)


def _fin_body(msc, mtc, psc, ptc, thr_ref, mx_ref):
    margins = msc[...] + mtc[...]
    lanes = lax.broadcasted_iota(jnp.int32, (_B, 16), 1)
    valid = lanes < _NA
    logits = margins * jnp.float32(1.0 / _T)
    mrow = jnp.max(jnp.where(valid, logits, jnp.float32(-1e30)),
                   axis=1, keepdims=True)
    e = jnp.where(valid, jnp.exp(logits - mrow), jnp.float32(0.0))
    thr_ref[...] = (e / jnp.sum(e, axis=1, keepdims=True))[:, :_NA]
    mx_ref[...] = jnp.full((1, 1),
                           jnp.maximum(jnp.max(psc[...]), jnp.max(ptc[...])))


_fin = pl.pallas_call(
    _fin_body,
    out_shape=(jax.ShapeDtypeStruct((_B, _NA), jnp.float32),
               jax.ShapeDtypeStruct((1, 1), jnp.float32)),
)


def kernel(outputs1, outputs2, outputs3, outputs4, outputs5, outputs6,
           outputs7, outputs8, mimic, targets, n_test):
    tgt32 = targets.astype(jnp.int32)
    marg_sc, part_sc = _sc_stage(outputs1, outputs2, outputs3, outputs4,
                                 tgt32)
    marg_tc, pm_tc = _tc_marg(outputs5, outputs6, outputs7, outputs8, mimic,
                              tgt32.reshape(_B, 1))
    thr, mx = _fin(marg_sc, marg_tc, part_sc, pm_tc)
    return mx.reshape(()), thr
